# initial kernel scaffold (unmeasured)
import jax
import jax.numpy as jnp
from jax import lax
from jax.experimental import pallas as pl
from jax.experimental.pallas import tpu as pltpu

N_DEV = 8


def kernel(x, w_mat):
    m, k = x.shape
    _, n = w_mat.shape
    m_per = m // N_DEV

    def body(x_ref, w_ref, out_ref, comm_ref, send_sems, recv_sems, credit_sems):
        my = lax.axis_index("i")
        left = lax.rem(my + N_DEV - 1, N_DEV)
        right = lax.rem(my + 1, N_DEV)

        barrier_sem = pltpu.get_barrier_semaphore()
        for nbr in (left, right):
            pl.semaphore_signal(
                barrier_sem, inc=1,
                device_id=(nbr,), device_id_type=pl.DeviceIdType.MESH,
            )
        pl.semaphore_wait(barrier_sem, 2)

        def partial(c):
            xc = x_ref[pl.ds(c * m_per, m_per), :]
            return jnp.dot(xc, w_ref[:, :], preferred_element_type=jnp.float32)

        comm_ref[0, :, :] = partial(left).astype(jnp.bfloat16)

        for s in range(N_DEV - 1):
            ss = s % 2
            rs = (s + 1) % 2
            if s >= 1:
                pl.semaphore_wait(credit_sems.at[(s - 1) % 2], 1)
            rdma = pltpu.make_async_remote_copy(
                src_ref=comm_ref.at[ss],
                dst_ref=comm_ref.at[rs],
                send_sem=send_sems.at[ss],
                recv_sem=recv_sems.at[rs],
                device_id=(right,),
                device_id_type=pl.DeviceIdType.MESH,
            )
            rdma.start()
            rdma.wait()
            c_r = lax.rem(my + 2 * N_DEV - s - 2, N_DEV)
            acc = comm_ref[rs, :, :].astype(jnp.float32) + partial(c_r)
            if s < N_DEV - 2:
                comm_ref[rs, :, :] = acc.astype(jnp.bfloat16)
                pl.semaphore_signal(
                    credit_sems.at[ss], inc=1,
                    device_id=(left,), device_id_type=pl.DeviceIdType.MESH,
                )
            else:
                out_ref[:, :] = acc

    return pl.pallas_call(
        body,
        out_shape=jax.ShapeDtypeStruct((m_per, n), jnp.float32),
        in_specs=[
            pl.BlockSpec(memory_space=pltpu.VMEM),
            pl.BlockSpec(memory_space=pltpu.VMEM),
        ],
        out_specs=pl.BlockSpec(memory_space=pltpu.VMEM),
        scratch_shapes=[
            pltpu.VMEM((2, m_per, n), jnp.bfloat16),
            pltpu.SemaphoreType.DMA((2,)),
            pltpu.SemaphoreType.DMA((2,)),
            pltpu.SemaphoreType.REGULAR((2,)),
        ],
        compiler_params=pltpu.CompilerParams(collective_id=0),
    )(x, w_mat)


# baseline (device time: 733469 ns/iter reference)
import jax
import jax.numpy as jnp
from jax import lax
from jax.experimental import pallas as pl
from jax.experimental.pallas import tpu as pltpu

N_DEV = 8


def kernel(x, w_mat):
    m, k = x.shape
    _, n = w_mat.shape
    m_per = m // N_DEV
    x = x.astype(jnp.bfloat16)
    w_mat = w_mat.astype(jnp.bfloat16)

    def body(x_ref, w_ref, out_ref, comm_ref, send_sems, recv_sems, credit_sems):
        my = lax.axis_index("i")
        left = lax.rem(my + N_DEV - 1, N_DEV)
        right = lax.rem(my + 1, N_DEV)

        barrier_sem = pltpu.get_barrier_semaphore()
        for nbr in (left, right):
            pl.semaphore_signal(
                barrier_sem, inc=1,
                device_id=(nbr,), device_id_type=pl.DeviceIdType.MESH,
            )
        pl.semaphore_wait(barrier_sem, 2)

        def partial(c):
            xc = x_ref[pl.ds(c * m_per, m_per), :]
            return jnp.dot(xc, w_ref[:, :], preferred_element_type=jnp.float32)

        comm_ref[0, :, :] = partial(left).astype(jnp.bfloat16)

        for s in range(N_DEV - 1):
            ss = s % 2
            rs = (s + 1) % 2
            if s >= 1:
                pl.semaphore_wait(credit_sems.at[(s - 1) % 2], 1)
            rdma = pltpu.make_async_remote_copy(
                src_ref=comm_ref.at[ss],
                dst_ref=comm_ref.at[rs],
                send_sem=send_sems.at[ss],
                recv_sem=recv_sems.at[rs],
                device_id=(right,),
                device_id_type=pl.DeviceIdType.MESH,
            )
            rdma.start()
            rdma.wait()
            c_r = lax.rem(my + 2 * N_DEV - s - 2, N_DEV)
            acc = comm_ref[rs, :, :].astype(jnp.float32) + partial(c_r)
            if s < N_DEV - 2:
                comm_ref[rs, :, :] = acc.astype(jnp.bfloat16)
                pl.semaphore_signal(
                    credit_sems.at[ss], inc=1,
                    device_id=(left,), device_id_type=pl.DeviceIdType.MESH,
                )
            else:
                out_ref[:, :] = acc

    return pl.pallas_call(
        body,
        out_shape=jax.ShapeDtypeStruct((m_per, n), jnp.float32),
        in_specs=[
            pl.BlockSpec(memory_space=pltpu.VMEM),
            pl.BlockSpec(memory_space=pltpu.VMEM),
        ],
        out_specs=pl.BlockSpec(memory_space=pltpu.VMEM),
        scratch_shapes=[
            pltpu.VMEM((2, m_per, n), jnp.bfloat16),
            pltpu.SemaphoreType.DMA((2,)),
            pltpu.SemaphoreType.DMA((2,)),
            pltpu.SemaphoreType.REGULAR((2,)),
        ],
        compiler_params=pltpu.CompilerParams(
            collective_id=0,
            vmem_limit_bytes=100 * 1024 * 1024,
        ),
    )(x, w_mat)


# device time: 393401 ns/iter; 1.8644x vs baseline; 1.8644x over previous
import jax
import jax.numpy as jnp
from jax import lax
from jax.experimental import pallas as pl
from jax.experimental.pallas import tpu as pltpu

N_DEV = 8


def kernel(x, w_mat):
    m, k = x.shape
    _, n = w_mat.shape
    m_per = m // N_DEV
    half = m_per // 2
    x = x.astype(jnp.bfloat16)
    w_mat = w_mat.astype(jnp.bfloat16)

    def body(x_ref, w_ref, out_ref, comm_r, comm_l,
             send_r, recv_r, send_l, recv_l, credit_r, credit_l):
        my = lax.axis_index("i")
        left = lax.rem(my + N_DEV - 1, N_DEV)
        right = lax.rem(my + 1, N_DEV)

        barrier_sem = pltpu.get_barrier_semaphore()
        for nbr in (left, right):
            pl.semaphore_signal(
                barrier_sem, inc=1,
                device_id=(nbr,), device_id_type=pl.DeviceIdType.MESH,
            )
        pl.semaphore_wait(barrier_sem, 2)

        def partial_top(c):
            xc = x_ref[pl.ds(c * m_per, half), :]
            return jnp.dot(xc, w_ref[:, :], preferred_element_type=jnp.float32)

        def partial_bot(c):
            xc = x_ref[pl.ds(c * m_per + half, half), :]
            return jnp.dot(xc, w_ref[:, :], preferred_element_type=jnp.float32)

        comm_r[0, :, :] = partial_top(left).astype(jnp.bfloat16)
        comm_l[0, :, :] = partial_bot(right).astype(jnp.bfloat16)

        for s in range(N_DEV - 1):
            ss = s % 2
            rs = (s + 1) % 2
            if s >= 1:
                pl.semaphore_wait(credit_r.at[(s - 1) % 2], 1)
                pl.semaphore_wait(credit_l.at[(s - 1) % 2], 1)
            rdma_r = pltpu.make_async_remote_copy(
                src_ref=comm_r.at[ss], dst_ref=comm_r.at[rs],
                send_sem=send_r.at[ss], recv_sem=recv_r.at[rs],
                device_id=(right,), device_id_type=pl.DeviceIdType.MESH,
            )
            rdma_l = pltpu.make_async_remote_copy(
                src_ref=comm_l.at[ss], dst_ref=comm_l.at[rs],
                send_sem=send_l.at[ss], recv_sem=recv_l.at[rs],
                device_id=(left,), device_id_type=pl.DeviceIdType.MESH,
            )
            rdma_r.start()
            rdma_l.start()
            c_top = lax.rem(my + 2 * N_DEV - s - 2, N_DEV)
            c_bot = lax.rem(my + s + 2, N_DEV)
            p_top = partial_top(c_top)
            p_bot = partial_bot(c_bot)
            rdma_r.wait()
            rdma_l.wait()
            acc_top = comm_r[rs, :, :].astype(jnp.float32) + p_top
            acc_bot = comm_l[rs, :, :].astype(jnp.float32) + p_bot
            if s < N_DEV - 2:
                comm_r[rs, :, :] = acc_top.astype(jnp.bfloat16)
                comm_l[rs, :, :] = acc_bot.astype(jnp.bfloat16)
                pl.semaphore_signal(
                    credit_r.at[ss], inc=1,
                    device_id=(left,), device_id_type=pl.DeviceIdType.MESH,
                )
                pl.semaphore_signal(
                    credit_l.at[ss], inc=1,
                    device_id=(right,), device_id_type=pl.DeviceIdType.MESH,
                )
            else:
                out_ref[pl.ds(0, half), :] = acc_top
                out_ref[pl.ds(half, half), :] = acc_bot

    return pl.pallas_call(
        body,
        out_shape=jax.ShapeDtypeStruct((m_per, n), jnp.float32),
        in_specs=[
            pl.BlockSpec(memory_space=pltpu.VMEM),
            pl.BlockSpec(memory_space=pltpu.VMEM),
        ],
        out_specs=pl.BlockSpec(memory_space=pltpu.VMEM),
        scratch_shapes=[
            pltpu.VMEM((2, half, n), jnp.bfloat16),
            pltpu.VMEM((2, half, n), jnp.bfloat16),
            pltpu.SemaphoreType.DMA((2,)),
            pltpu.SemaphoreType.DMA((2,)),
            pltpu.SemaphoreType.DMA((2,)),
            pltpu.SemaphoreType.DMA((2,)),
            pltpu.SemaphoreType.REGULAR((2,)),
            pltpu.SemaphoreType.REGULAR((2,)),
        ],
        compiler_params=pltpu.CompilerParams(
            collective_id=0,
            vmem_limit_bytes=100 * 1024 * 1024,
        ),
    )(x, w_mat)


# device time: 379130 ns/iter; 1.9346x vs baseline; 1.0376x over previous
import jax
import jax.numpy as jnp
from jax import lax
from jax.experimental import pallas as pl
from jax.experimental.pallas import tpu as pltpu

N_DEV = 8
SUB = 2


def kernel(x, w_mat):
    m, k = x.shape
    _, n = w_mat.shape
    m_per = m // N_DEV
    half = m_per // 2
    rows = half // SUB
    x = x.astype(jnp.bfloat16)
    w_mat = w_mat.astype(jnp.bfloat16)

    def body(x_ref, w_ref, out_ref, comm_r, comm_l,
             send_r, recv_r, send_l, recv_l, credit_r, credit_l):
        my = lax.axis_index("i")
        left = lax.rem(my + N_DEV - 1, N_DEV)
        right = lax.rem(my + 1, N_DEV)

        barrier_sem = pltpu.get_barrier_semaphore()
        for nbr in (left, right):
            pl.semaphore_signal(
                barrier_sem, inc=1,
                device_id=(nbr,), device_id_type=pl.DeviceIdType.MESH,
            )
        pl.semaphore_wait(barrier_sem, 2)

        def partial_top(c):
            xc = x_ref[pl.ds(c * m_per, half), :]
            return jnp.dot(xc, w_ref[:, :], preferred_element_type=jnp.float32)

        def partial_bot(c):
            xc = x_ref[pl.ds(c * m_per + half, half), :]
            return jnp.dot(xc, w_ref[:, :], preferred_element_type=jnp.float32)

        def sub_copy(comm, slot_s, slot_r, j, ssem, rsem, dev):
            return pltpu.make_async_remote_copy(
                src_ref=comm.at[slot_s, pl.ds(j * rows, rows), :],
                dst_ref=comm.at[slot_r, pl.ds(j * rows, rows), :],
                send_sem=ssem, recv_sem=rsem,
                device_id=(dev,), device_id_type=pl.DeviceIdType.MESH,
            )

        comm_r[0, :, :] = partial_top(left).astype(jnp.bfloat16)
        comm_l[0, :, :] = partial_bot(right).astype(jnp.bfloat16)

        p_top = p_bot = None
        for s in range(N_DEV - 1):
            ss = s % 2
            rs = (s + 1) % 2
            if s >= 1:
                pl.semaphore_wait(credit_r.at[(s - 1) % 2], 1)
                pl.semaphore_wait(credit_l.at[(s - 1) % 2], 1)
            sends = []
            for j in range(SUB):
                if s >= 1:
                    sub_copy(comm_r, ss, ss, j, send_r.at[ss, j],
                             recv_r.at[ss, j], left).wait_recv()
                    comm_r[ss, pl.ds(j * rows, rows), :] = (
                        comm_r[ss, pl.ds(j * rows, rows), :].astype(jnp.float32)
                        + p_top[j * rows:(j + 1) * rows, :]
                    ).astype(jnp.bfloat16)
                    sub_copy(comm_l, ss, ss, j, send_l.at[ss, j],
                             recv_l.at[ss, j], right).wait_recv()
                    comm_l[ss, pl.ds(j * rows, rows), :] = (
                        comm_l[ss, pl.ds(j * rows, rows), :].astype(jnp.float32)
                        + p_bot[j * rows:(j + 1) * rows, :]
                    ).astype(jnp.bfloat16)
                r = sub_copy(comm_r, ss, rs, j, send_r.at[ss, j],
                             recv_r.at[rs, j], right)
                l = sub_copy(comm_l, ss, rs, j, send_l.at[ss, j],
                             recv_l.at[rs, j], left)
                r.start()
                l.start()
                sends.append((r, l))
            p_top = partial_top(lax.rem(my + 2 * N_DEV - s - 2, N_DEV))
            p_bot = partial_bot(lax.rem(my + s + 2, N_DEV))
            for r, l in sends:
                r.wait_send()
                l.wait_send()
            if s < N_DEV - 2:
                pl.semaphore_signal(
                    credit_r.at[ss], inc=1,
                    device_id=(left,), device_id_type=pl.DeviceIdType.MESH,
                )
                pl.semaphore_signal(
                    credit_l.at[ss], inc=1,
                    device_id=(right,), device_id_type=pl.DeviceIdType.MESH,
                )

        for j in range(SUB):
            sub_copy(comm_r, 1, 1, j, send_r.at[1, j],
                     recv_r.at[1, j], left).wait_recv()
            out_ref[pl.ds(j * rows, rows), :] = (
                comm_r[1, pl.ds(j * rows, rows), :].astype(jnp.float32)
                + p_top[j * rows:(j + 1) * rows, :]
            )
            sub_copy(comm_l, 1, 1, j, send_l.at[1, j],
                     recv_l.at[1, j], right).wait_recv()
            out_ref[pl.ds(half + j * rows, rows), :] = (
                comm_l[1, pl.ds(j * rows, rows), :].astype(jnp.float32)
                + p_bot[j * rows:(j + 1) * rows, :]
            )

    return pl.pallas_call(
        body,
        out_shape=jax.ShapeDtypeStruct((m_per, n), jnp.float32),
        in_specs=[
            pl.BlockSpec(memory_space=pltpu.VMEM),
            pl.BlockSpec(memory_space=pltpu.VMEM),
        ],
        out_specs=pl.BlockSpec(memory_space=pltpu.VMEM),
        scratch_shapes=[
            pltpu.VMEM((2, half, n), jnp.bfloat16),
            pltpu.VMEM((2, half, n), jnp.bfloat16),
            pltpu.SemaphoreType.DMA((2, SUB)),
            pltpu.SemaphoreType.DMA((2, SUB)),
            pltpu.SemaphoreType.DMA((2, SUB)),
            pltpu.SemaphoreType.DMA((2, SUB)),
            pltpu.SemaphoreType.REGULAR((2,)),
            pltpu.SemaphoreType.REGULAR((2,)),
        ],
        compiler_params=pltpu.CompilerParams(
            collective_id=0,
            vmem_limit_bytes=100 * 1024 * 1024,
        ),
    )(x, w_mat)
